# baseline (speedup 1.0000x reference)
import functools

import jax
import jax.numpy as jnp
from jax import lax
from jax.experimental import pallas as pl
from jax.experimental.pallas import tpu as pltpu
from jax.experimental.pallas import tpu_sc as plsc

VOCAB = 49408
EMBED = 768
SEQ = 77
BATCH = 1024
FLAT = BATCH * SEQ
CHUNK = 11
CPS = SEQ // CHUNK
NCHUNKS = FLAT // CHUNK
NBUF = 4
LANES = 16
PREF = 2


def _build_sc_call():
    mesh = plsc.VectorSubcoreMesh(core_axis_name="c", subcore_axis_name="s")
    nw = mesh.num_cores * mesh.num_subcores
    nch_w = NCHUNKS // nw

    scratch = (
        [pltpu.VMEM((nch_w, CHUNK), jnp.int32)]
        + [pltpu.VMEM((SEQ, EMBED), jnp.float32)]
        + [pltpu.VMEM((CHUNK, EMBED), jnp.float32) for _ in range(NBUF)]
        + [pltpu.SemaphoreType.DMA for _ in range(2 * NBUF)]
    )

    @functools.partial(
        pl.kernel,
        out_type=jax.ShapeDtypeStruct((FLAT, EMBED), jnp.float32),
        mesh=mesh,
        scratch_types=scratch,
        compiler_params=pltpu.CompilerParams(use_tc_tiling_on_sc=False),
    )
    def sc_embed(table, idx, pos, out, idx_v, pos_v,
                 b0, b1, b2, b3, sg0, sg1, sg2, sg3, ss0, ss1, ss2, ss3):
        bufs = (b0, b1, b2, b3)
        sg = (sg0, sg1, sg2, sg3)
        ss = (ss0, ss1, ss2, ss3)
        wid = lax.axis_index("s") * mesh.num_cores + lax.axis_index("c")
        cbase = wid * nch_w

        pltpu.sync_copy(idx.at[pl.ds(cbase, nch_w)], idx_v)
        pltpu.sync_copy(pos, pos_v)

        def gather_issue(k, b):
            pltpu.async_copy(table.at[idx_v.at[k]], bufs[b], sg[b])

        def gather_wait(b):
            pltpu.make_async_copy(table.at[idx_v.at[0]], bufs[b], sg[b]).wait()

        def scatter_issue(k, b):
            pltpu.async_copy(
                bufs[b], out.at[pl.ds((cbase + k) * CHUNK, CHUNK)], ss[b])

        def scatter_wait(b):
            pltpu.make_async_copy(
                bufs[b], out.at[pl.ds(cbase * CHUNK, CHUNK)], ss[b]).wait()

        def add_pos(k, b):
            poff = lax.rem(k, CPS) * CHUNK

            @pl.loop(0, CHUNK)
            def _row(r):
                for j in range(EMBED // LANES):
                    x = pos_v[poff + r, pl.ds(j * LANES, LANES)]
                    plsc.addupdate(bufs[b].at[r, pl.ds(j * LANES, LANES)], x)

        for p in range(PREF):
            gather_issue(p, p)

        @pl.loop(0, nch_w, step=NBUF)
        def _group(g):
            for b in range(NBUF):
                k = g + b
                gather_wait(b)
                add_pos(k, b)
                scatter_issue(k, b)
                m = k + PREF
                bm = (b + PREF) % NBUF

                @pl.when(m < nch_w)
                def _():
                    @pl.when(m >= NBUF)
                    def _():
                        scatter_wait(bm)

                    gather_issue(m, bm)

        for b in range(NBUF):
            scatter_wait(b)

    return sc_embed


def kernel(tokens, token_table, positional_embedding):
    idx = tokens.astype(jnp.int32).reshape(NCHUNKS, CHUNK)
    out = _build_sc_call()(token_table, idx, positional_embedding)
    return out.reshape(BATCH, SEQ, EMBED)


# speedup vs baseline: 1.6836x; 1.6836x over previous
import functools

import jax
import jax.numpy as jnp
from jax import lax
from jax.experimental import pallas as pl
from jax.experimental.pallas import tpu as pltpu
from jax.experimental.pallas import tpu_sc as plsc

VOCAB = 49408
EMBED = 768
SEQ = 77
SEQP = 80
BATCH = 1024
LANES = 16
CHUNK = 40
STRIPE = 8
NST = CHUNK // STRIPE
PIECES = EMBED // 128


def _build_sc_call():
    mesh = plsc.VectorSubcoreMesh(core_axis_name="c", subcore_axis_name="s")
    nw = mesh.num_cores * mesh.num_subcores
    seq_w = BATCH // nw

    scratch = (
        [pltpu.VMEM((seq_w, SEQP), jnp.int32)]
        + [pltpu.VMEM((SEQ * PIECES, 128), jnp.float32)]
        + [pltpu.VMEM((CHUNK, EMBED), jnp.float32) for _ in range(2)]
        + [pltpu.VMEM((1, 48), jnp.int32)]
        + [pltpu.SemaphoreType.DMA for _ in range(4)]
    )

    @functools.partial(
        pl.kernel,
        out_type=jax.ShapeDtypeStruct((BATCH, SEQ, EMBED), jnp.float32),
        mesh=mesh,
        scratch_types=scratch,
    )
    def sc_embed(tokens, table, pos, out, idx_v, pos_v,
                 buf0, buf1, rows_v, sg0, sg1, ss0, ss1):
        bufs = (buf0, buf1)
        sg = (sg0, sg1)
        ss = (ss0, ss1)
        wid = lax.axis_index("s") * mesh.num_cores + lax.axis_index("c")
        sbase = wid * seq_w

        pltpu.sync_copy(tokens.at[pl.ds(sbase, seq_w)], idx_v)
        pltpu.sync_copy(pos, pos_v)
        for t in range(3):
            rows_v[0, pl.ds(t * LANES, LANES)] = (
                lax.iota(jnp.int32, LANES) + (CHUNK + t * LANES))

        def gather_issue(s, h):
            pltpu.async_copy(
                table.at[idx_v.at[s, pl.ds(h * CHUNK, CHUNK)]],
                bufs[h], sg[h])

        def gather_wait(h):
            pltpu.make_async_copy(
                table.at[idx_v.at[0, pl.ds(h * CHUNK, CHUNK)]],
                bufs[h], sg[h]).wait()

        def _scatter_dst(s, h):
            if h == 0:
                return out.at[sbase + s, pl.ds(0, CHUNK)]
            return out.at[sbase + s].at[rows_v.at[0, pl.ds(0, CHUNK)]]

        def scatter_issue(s, h):
            pltpu.async_copy(bufs[h], _scatter_dst(s, h), ss[h])

        def scatter_wait(h):
            pltpu.make_async_copy(bufs[h], _scatter_dst(0, h), ss[h]).wait()

        def add_pos(h):
            @pl.loop(0, NST)
            def _stripe(st):
                row0 = h * CHUNK + st * STRIPE
                for r in range(STRIPE):
                    prow_log = jnp.minimum(row0 + r, SEQ - 1) * PIECES
                    for p in range(PIECES):
                        for q in range(128 // LANES):
                            w = p * 1024 + r * 128 + q * LANES
                            x = pos_v[prow_log + p, pl.ds(q * LANES, LANES)]
                            plsc.addupdate(
                                bufs[h].at[st * STRIPE + w // EMBED,
                                           pl.ds(w % EMBED, LANES)], x)

        gather_issue(0, 0)

        @pl.loop(0, seq_w)
        def _seq(s):
            gather_wait(0)

            @pl.when(s > 0)
            def _():
                scatter_wait(1)

            gather_issue(s, 1)
            add_pos(0)
            scatter_issue(s, 0)

            gather_wait(1)
            scatter_wait(0)

            @pl.when(s < seq_w - 1)
            def _():
                gather_issue(s + 1, 0)

            add_pos(1)
            scatter_issue(s, 1)

        scatter_wait(1)

    return sc_embed


def kernel(tokens, token_table, positional_embedding):
    tok = jnp.pad(tokens.astype(jnp.int32), ((0, 0), (0, SEQP - SEQ)))
    pos2 = positional_embedding.reshape(SEQ * PIECES, 128)
    return _build_sc_call()(tok, token_table, pos2)


# speedup vs baseline: 4.1656x; 2.4742x over previous
import functools

import jax
import jax.numpy as jnp
from jax import lax
from jax.experimental import pallas as pl
from jax.experimental.pallas import tpu as pltpu
from jax.experimental.pallas import tpu_sc as plsc

VOCAB = 49408
EMBED = 768
SEQ = 77
BATCH = 1024
LANES = 16
STRIPE = 8
PIECES = EMBED // 128
NBUF = 4
CHUNK = 16
NW = 32


def _build_sc_call():
    mesh = plsc.VectorSubcoreMesh(core_axis_name="c", subcore_axis_name="s")
    bat_w = BATCH // NW
    idx_w = SEQ * bat_w
    nchunks = idx_w // CHUNK
    vchunks = -(-nchunks // NBUF) * NBUF
    nst = CHUNK // STRIPE

    scratch = (
        [pltpu.VMEM((idx_w,), jnp.int32)]
        + [pltpu.VMEM((SEQ * PIECES, 128), jnp.float32)]
        + [pltpu.VMEM((CHUNK, EMBED), jnp.float32) for _ in range(NBUF)]
        + [pltpu.SemaphoreType.DMA for _ in range(2 * NBUF)]
    )

    @functools.partial(
        pl.kernel,
        out_type=jax.ShapeDtypeStruct((SEQ, BATCH, EMBED), jnp.float32),
        mesh=mesh,
        scratch_types=scratch,
    )
    def sc_embed(tokens, table, pos, out, idx_v, pos_v,
                 buf0, buf1, buf2, buf3, sg0, sg1, sg2, sg3,
                 ss0, ss1, ss2, ss3):
        bufs = (buf0, buf1, buf2, buf3)
        sg = (sg0, sg1, sg2, sg3)
        ss = (ss0, ss1, ss2, ss3)
        wid = lax.axis_index("s") * mesh.num_cores + lax.axis_index("c")
        bbase = wid * bat_w

        pltpu.sync_copy(tokens.at[wid], idx_v)
        pltpu.sync_copy(pos, pos_v)

        def split(c):
            cc = jnp.minimum(c, nchunks - 1)
            return cc // 2, cc

        def gather_issue(c, b):
            _, cc = split(c)
            pltpu.async_copy(
                table.at[idx_v.at[pl.ds(cc * CHUNK, CHUNK)]], bufs[b], sg[b])

        def gather_wait(b):
            pltpu.make_async_copy(
                table.at[idx_v.at[pl.ds(0, CHUNK)]], bufs[b], sg[b]).wait()

        def scatter_issue(c, b):
            t, cc = split(c)
            off = bbase + (cc % 2) * CHUNK
            pltpu.async_copy(bufs[b], out.at[t, pl.ds(off, CHUNK)], ss[b])

        def scatter_wait(b):
            pltpu.make_async_copy(
                bufs[b], out.at[0, pl.ds(bbase, CHUNK)], ss[b]).wait()

        def add_pos(c, b):
            t, _ = split(c)
            prow = t * PIECES

            @pl.loop(0, nst)
            def _stripe(st):
                for p in range(PIECES):
                    for q in range(128 // LANES):
                        x = pos_v[prow + p, pl.ds(q * LANES, LANES)]
                        for r in range(STRIPE):
                            w = p * 1024 + r * 128 + q * LANES
                            plsc.addupdate(
                                bufs[b].at[st * STRIPE + w // EMBED,
                                           pl.ds(w % EMBED, LANES)], x)

        gather_issue(0, 0)
        gather_issue(1, 1)

        @pl.loop(0, vchunks, step=NBUF)
        def _group(g):
            for i in range(NBUF):
                c = g + i
                gather_wait(i)
                add_pos(c, i)
                scatter_issue(c, i)

                @pl.when(c >= 2)
                def _():
                    scatter_wait((i + 2) % NBUF)

                @pl.when(c + 2 < vchunks)
                def _():
                    gather_issue(c + 2, (i + 2) % NBUF)

        scatter_wait((vchunks - 2) % NBUF)
        scatter_wait((vchunks - 1) % NBUF)

    return sc_embed


def kernel(tokens, token_table, positional_embedding):
    tok = (tokens.astype(jnp.int32)
           .reshape(NW, BATCH // NW, SEQ)
           .transpose(0, 2, 1)
           .reshape(NW, SEQ * (BATCH // NW)))
    pos2 = positional_embedding.reshape(SEQ * PIECES, 128)
    out = _build_sc_call()(tok, token_table, pos2)
    return out.transpose(1, 0, 2)
